# baseline (speedup 1.0000x reference)
import functools

import jax
import jax.numpy as jnp
from jax import lax
from jax.experimental import pallas as pl
from jax.experimental.pallas import tpu as pltpu
from jax.experimental.pallas import tpu_sc as plsc

N = 10000
NP = 10240
E = 160000
NC = 2
NS = 16
K = 128
EPT = E // NS
EPTP = 10240
NCH = EPTP // K
CPM = 8
NM = NCH // CPM
RPT = NP // NS


@functools.lru_cache(maxsize=None)
def _make_seg_sum(nb):
    nbc = nb // NC

    mesh = plsc.VectorSubcoreMesh(
        core_axis_name="c", subcore_axis_name="s",
        num_cores=NC, num_subcores=NS)

    @functools.partial(
        pl.kernel,
        out_type=[jax.ShapeDtypeStruct((NP, 128), jnp.float32)
                  for _ in range(nb)],
        mesh=mesh,
        scratch_types=[
            pltpu.VMEM_SHARED((NP, 128), jnp.float32),
            pltpu.VMEM((NCH, K), jnp.int32),
            pltpu.VMEM((CPM, K), jnp.int32),
            pltpu.VMEM((CPM, K), jnp.int32),
            pltpu.VMEM((K, 128), jnp.float32),
            pltpu.VMEM((K, 128), jnp.float32),
            pltpu.SemaphoreType.DMA,
            pltpu.SemaphoreType.DMA,
            pltpu.SemaphoreType.DMA,
            pltpu.SemaphoreType.DMA,
            pltpu.SemaphoreType.DMA,
            pltpu.SemaphoreType.DMA,
        ],
    )
    def seg(src_hbm, dst_hbm, *rest):
        feats = rest[:nb]
        outs = rest[nb:2 * nb]
        (acc, sidx, db0, db1, r0, r1,
         si0, si1, sg0, sg1, ss0, ss1) = rest[2 * nb:]
        dbuf = (db0, db1)
        rbuf = (r0, r1)
        si = (si0, si1)
        sg = (sg0, sg1)
        ss = (ss0, ss1)
        s = lax.axis_index("s")
        c = lax.axis_index("c")
        row0 = s * RPT
        zv = jnp.zeros((16,), jnp.float32)

        pltpu.sync_copy(src_hbm.at[s], sidx)

        def fill_r0_zero():
            @pl.loop(0, K)
            def _(r):
                for j in range(8):
                    r0[r, pl.ds(j * 16, 16)] = zv

        def zero_stripe():
            for z in range(RPT // K):
                pltpu.sync_copy(r0, acc.at[pl.ds(row0 + z * K, K)])

        fill_r0_zero()
        zero_stripe()

        def im_start(m, msl):
            pltpu.async_copy(dst_hbm.at[s, m], dbuf[msl], si[msl])

        def im_wait(msl):
            pltpu.make_async_copy(dst_hbm.at[s, 0], dbuf[msl],
                                  si[msl]).wait()

        def do_pass(feat, out, last):
            plsc.subcore_barrier()

            def g_start(i, sl):
                pltpu.async_copy(feat.at[sidx.at[i]], rbuf[sl], sg[sl])

            def g_wait(sl):
                pltpu.make_async_copy(feat.at[sidx.at[0]], rbuf[sl],
                                      sg[sl]).wait()

            def s_start(k, msl, sl):
                pltpu.async_copy(rbuf[sl], acc.at[dbuf[msl].at[k]], ss[sl],
                                 add=True)

            def s_wait(sl):
                pltpu.make_async_copy(rbuf[sl], acc.at[dbuf[0].at[0]],
                                      ss[sl]).wait()

            def chunk_step(m, k, msl, c_sm2, c_m1):
                sl = k % 2
                if c_sm2 is True:
                    s_wait(sl)
                elif c_sm2 is not False:
                    pl.when(c_sm2)(lambda: s_wait(sl))
                g_start(m * CPM + k, sl)

                def _retire():
                    g_wait(1 - sl)
                    if k >= 1:
                        s_start(k - 1, msl, 1 - sl)
                    else:
                        s_start(CPM - 1, 1 - msl, 1 - sl)
                if c_m1 is True:
                    _retire()
                elif c_m1 is not False:
                    pl.when(c_m1)(_retire)

            im_start(0, 0)

            @pl.loop(0, NM // 2)
            def _(t):
                for half in range(2):
                    m = 2 * t + half
                    msl = half
                    mge1 = True if half == 1 else (t >= 1)
                    im_wait(msl)
                    chunk_step(m, 0, msl, mge1, mge1)
                    chunk_step(m, 1, msl, mge1, True)
                    if half == 0:
                        im_start(2 * t + 1, 1)
                    else:
                        pl.when(t < NM // 2 - 1)(
                            lambda: im_start(2 * t + 2, 0))
                    for k in range(2, CPM):
                        chunk_step(m, k, msl, True, True)

            g_wait(1)
            s_start(CPM - 1, 1, 1)
            s_wait(0)
            s_wait(1)

            plsc.subcore_barrier()
            pltpu.sync_copy(acc.at[pl.ds(row0, RPT)],
                            out.at[pl.ds(row0, RPT)])
            if not last:
                fill_r0_zero()
                zero_stripe()

        for core in range(NC):
            for bi in range(nbc):
                blk = core * nbc + bi

                @pl.when(c == core)
                def _(blk=blk, bi=bi):
                    do_pass(feats[blk], outs[blk], bi == nbc - 1)

    return seg


BM = 1280


def _mm1_body(a0, a1, w, b, o0, o1, o2, o3):
    acc = jnp.dot(a0[...], w[0:128, :], preferred_element_type=jnp.float32)
    acc += jnp.dot(a1[...], w[128:256, :], preferred_element_type=jnp.float32)
    h = jnp.maximum(acc + b[...], 0.0)
    o0[...] = h[:, 0:128]
    o1[...] = h[:, 128:256]
    o2[...] = h[:, 256:384]
    o3[...] = h[:, 384:512]


def _mm1(a0, a1, w, b):
    grid = (NP // BM,)
    blk = lambda i: (i, 0)
    cst = lambda i: (0, 0)
    return pl.pallas_call(
        _mm1_body,
        grid=grid,
        in_specs=[
            pl.BlockSpec((BM, 128), blk),
            pl.BlockSpec((BM, 128), blk),
            pl.BlockSpec((256, 512), cst),
            pl.BlockSpec((1, 512), cst),
        ],
        out_specs=[pl.BlockSpec((BM, 128), blk) for _ in range(4)],
        out_shape=[jax.ShapeDtypeStruct((NP, 128), jnp.float32)
                   for _ in range(4)],
    )(a0, a1, w, b)


def _mm2_body(g0, g1, g2, g3, w, b, wm1, bm1, wm2, bm2, out, sums):
    acc = jnp.dot(g0[...], w[0:128, :], preferred_element_type=jnp.float32)
    acc += jnp.dot(g1[...], w[128:256, :], preferred_element_type=jnp.float32)
    acc += jnp.dot(g2[...], w[256:384, :], preferred_element_type=jnp.float32)
    acc += jnp.dot(g3[...], w[384:512, :], preferred_element_type=jnp.float32)
    h = jnp.maximum(acc + b[...], 0.0)
    row = (pl.program_id(0) * BM
           + lax.broadcasted_iota(jnp.int32, h.shape, 0))
    h = jnp.where(row < N, h, 0.0)
    cs = jnp.sum(h, axis=0, keepdims=True)

    @pl.when(pl.program_id(0) == 0)
    def _():
        sums[...] = jnp.zeros_like(sums)

    sums[...] += cs

    @pl.when(pl.program_id(0) == NP // BM - 1)
    def _():
        hg = sums[...] * (1.0 / N)
        m = jnp.maximum(
            jnp.dot(hg, wm1[...], preferred_element_type=jnp.float32)
            + bm1[...], 0.0)
        out[...] = jnp.maximum(
            jnp.dot(m, wm2[...], preferred_element_type=jnp.float32)
            + bm2[...], 0.0)


def _mm2_mlp(g0, g1, g2, g3, w, b, wm1, bm1, wm2, bm2):
    grid = (NP // BM,)
    blk = lambda i: (i, 0)
    cst = lambda i: (0, 0)
    return pl.pallas_call(
        _mm2_body,
        grid=grid,
        in_specs=[
            pl.BlockSpec((BM, 128), blk),
            pl.BlockSpec((BM, 128), blk),
            pl.BlockSpec((BM, 128), blk),
            pl.BlockSpec((BM, 128), blk),
            pl.BlockSpec((512, 512), cst),
            pl.BlockSpec((1, 512), cst),
            pl.BlockSpec((512, 512), cst),
            pl.BlockSpec((1, 512), cst),
            pl.BlockSpec((512, 256), cst),
            pl.BlockSpec((1, 256), cst),
        ],
        out_specs=pl.BlockSpec((1, 256), cst),
        out_shape=jax.ShapeDtypeStruct((1, 256), jnp.float32),
        scratch_shapes=[pltpu.VMEM((1, 512), jnp.float32)],
    )(g0, g1, g2, g3, w, b, wm1, bm1, wm2, bm2)


def kernel(x, edge_index, W1, b1, W2, b2, Wm1, bm1, Wm2, bm2):
    pad = EPTP - EPT
    pad_src = jnp.arange(pad, dtype=jnp.int32) * 37 % N
    src4 = jnp.concatenate(
        [edge_index[0].reshape(NS, EPT),
         jnp.broadcast_to(pad_src, (NS, pad))],
        axis=1).reshape(NS, NCH, K)
    pad_dst = N + (jnp.arange(pad, dtype=jnp.int32) % (NP - N))
    dst4 = jnp.concatenate(
        [edge_index[1].reshape(NS, EPT),
         jnp.broadcast_to(pad_dst, (NS, pad))],
        axis=1).reshape(NS, NM, CPM, K)
    x0 = x[:, 0:128]
    x1 = x[:, 128:256]

    a0, a1 = _make_seg_sum(2)(src4, dst4, x0, x1)
    h0, h1, h2, h3 = _mm1(a0, a1, W1, b1.reshape(1, 512))
    g0, g1, g2, g3 = _make_seg_sum(4)(src4, dst4, h0, h1, h2, h3)
    return _mm2_mlp(g0, g1, g2, g3, W2, b2.reshape(1, 512),
                    Wm1, bm1.reshape(1, 512), Wm2, bm2.reshape(1, 256))
